# baseline (speedup 1.0000x reference)
import functools

import jax
import jax.numpy as jnp
from jax import lax
from jax.experimental import pallas as pl
from jax.experimental.pallas import tpu as pltpu
from jax.experimental.pallas import tpu_sc as plsc

D = 64
B = 16384
NC = 2
NS = 16
NW = NC * NS
BPW = B // NW
L = 16


def _sc_body(uemb, iemb, ubt, ibt, uidx, iidx,
             part_out, bsum_out,
             idx_u, idx_i, rows_u, rows_i, bu, bi, accv,
             sem_u, sem_i, sem_bu, sem_bi):
    c = lax.axis_index("c")
    s = lax.axis_index("s")
    wid = s * NC + c
    base = wid * BPW

    pltpu.sync_copy(uidx.at[pl.ds(base, BPW)], idx_u)
    pltpu.sync_copy(iidx.at[pl.ds(base, BPW)], idx_i)

    cp_u = pltpu.async_copy(uemb.at[idx_u], rows_u, sem_u)
    cp_i = pltpu.async_copy(iemb.at[idx_i], rows_i, sem_i)
    cp_bu = pltpu.async_copy(ubt.at[idx_u], bu, sem_bu)
    cp_bi = pltpu.async_copy(ibt.at[idx_i], bi, sem_bi)
    cp_u.wait()
    cp_i.wait()

    zero = jnp.zeros((L,), jnp.float32)

    def body(r0, accs):
        a0, a1, a2, a3 = accs
        for rr in range(4):
            r = r0 * 4 + rr
            a0 = a0 + rows_u[r, pl.ds(0, L)] * rows_i[r, pl.ds(0, L)]
            a1 = a1 + rows_u[r, pl.ds(L, L)] * rows_i[r, pl.ds(L, L)]
            a2 = a2 + rows_u[r, pl.ds(2 * L, L)] * rows_i[r, pl.ds(2 * L, L)]
            a3 = a3 + rows_u[r, pl.ds(3 * L, L)] * rows_i[r, pl.ds(3 * L, L)]
        return (a0, a1, a2, a3)

    a0, a1, a2, a3 = lax.fori_loop(0, BPW // 4, body, (zero, zero, zero, zero))
    accv[...] = (a0 + a1) + (a2 + a3)
    pltpu.sync_copy(accv, part_out.at[wid])

    cp_bu.wait()
    cp_bi.wait()
    for j in range(BPW // L):
        sl = pl.ds(j * L, L)
        bu[sl] = bu[sl] + bi[sl]
    pltpu.sync_copy(bu, bsum_out.at[pl.ds(base, BPW)])


_sc_call = functools.partial(
    pl.kernel,
    out_type=(
        jax.ShapeDtypeStruct((NW, L), jnp.float32),
        jax.ShapeDtypeStruct((B,), jnp.float32),
    ),
    mesh=plsc.VectorSubcoreMesh(core_axis_name="c", subcore_axis_name="s"),
    compiler_params=pltpu.CompilerParams(use_tc_tiling_on_sc=False),
    scratch_types=[
        pltpu.VMEM((BPW,), jnp.int32),
        pltpu.VMEM((BPW,), jnp.int32),
        pltpu.VMEM((BPW, D), jnp.float32),
        pltpu.VMEM((BPW, D), jnp.float32),
        pltpu.VMEM((BPW,), jnp.float32),
        pltpu.VMEM((BPW,), jnp.float32),
        pltpu.VMEM((L,), jnp.float32),
        pltpu.SemaphoreType.DMA,
        pltpu.SemaphoreType.DMA,
        pltpu.SemaphoreType.DMA,
        pltpu.SemaphoreType.DMA,
    ],
)(_sc_body)


def _finish_body(part_ref, bsum_ref, out_ref):
    scalar = jnp.sum(part_ref[...])
    out_ref[...] = jnp.maximum(bsum_ref[...] + scalar, 0.0)


def kernel(user_emb, user_bias_tbl, item_emb, item_bias_tbl, inputs):
    idx = inputs.astype(jnp.int32).T
    partials, bsum = _sc_call(
        user_emb, item_emb,
        user_bias_tbl.reshape(-1), item_bias_tbl.reshape(-1),
        idx[0], idx[1],
    )
    out = pl.pallas_call(
        _finish_body,
        out_shape=jax.ShapeDtypeStruct((B // 128, 128), jnp.float32),
    )(partials, bsum.reshape(B // 128, 128))
    return out.reshape(B, 1)


# speedup vs baseline: 1.4150x; 1.4150x over previous
import functools

import jax
import jax.numpy as jnp
from jax import lax
from jax.experimental import pallas as pl
from jax.experimental.pallas import tpu as pltpu
from jax.experimental.pallas import tpu_sc as plsc

D = 64
B = 16384
NC = 2
NS = 16
NW = NC * NS
BPW = B // NW
L = 16
NCH = BPW // L
TS = 8


def _dot_body(uemb, iemb, uidx, iidx, part_out, *refs):
    idx_u, idx_i = refs[0], refs[1]
    tu = refs[2:2 + L]
    ti = refs[2 + L:2 + 2 * L]
    accv = refs[2 + 2 * L]
    sems = refs[3 + 2 * L:]

    c = lax.axis_index("c")
    s = lax.axis_index("s")
    wid = s * NC + c
    base = wid * BPW

    pltpu.sync_copy(uidx.at[pl.ds(base, BPW)], idx_u)
    pltpu.sync_copy(iidx.at[pl.ds(base, BPW)], idx_i)

    def enqueue(r_u, r_i, j):
        t8u = pl.multiple_of((r_u // TS) * TS, TS)
        t8i = pl.multiple_of((r_i // TS) * TS, TS)
        pltpu.make_async_copy(uemb.at[pl.ds(t8u, TS), :], tu[j], sems[j]).start()
        pltpu.make_async_copy(iemb.at[pl.ds(t8i, TS), :], ti[j], sems[j]).start()

    def wait_slot(j):
        pltpu.make_async_copy(uemb.at[pl.ds(0, TS), :], tu[j], sems[j]).wait()
        pltpu.make_async_copy(iemb.at[pl.ds(0, TS), :], ti[j], sems[j]).wait()

    def consume(pvu, pvi, j, accs):
        a0, a1, a2, a3 = accs
        wait_slot(j)
        r_u = pvu[j]
        r_i = pvi[j]
        off_u = r_u - (r_u // TS) * TS
        off_i = r_i - (r_i // TS) * TS
        a0 = a0 + tu[j][off_u, pl.ds(0, L)] * ti[j][off_i, pl.ds(0, L)]
        a1 = a1 + tu[j][off_u, pl.ds(L, L)] * ti[j][off_i, pl.ds(L, L)]
        a2 = a2 + tu[j][off_u, pl.ds(2 * L, L)] * ti[j][off_i, pl.ds(2 * L, L)]
        a3 = a3 + tu[j][off_u, pl.ds(3 * L, L)] * ti[j][off_i, pl.ds(3 * L, L)]
        return (a0, a1, a2, a3)

    vu0 = idx_u[pl.ds(0, L)]
    vi0 = idx_i[pl.ds(0, L)]
    for j in range(L):
        enqueue(vu0[j], vi0[j], j)

    zero = jnp.zeros((L,), jnp.float32)

    def chunk_body(g, carry):
        a0, a1, a2, a3, pvu, pvi = carry
        vu = idx_u[pl.ds(g * L, L)]
        vi = idx_i[pl.ds(g * L, L)]
        accs = (a0, a1, a2, a3)
        for j in range(L):
            accs = consume(pvu, pvi, j, accs)
            enqueue(vu[j], vi[j], j)
        return accs + (vu, vi)

    a0, a1, a2, a3, pvu, pvi = lax.fori_loop(
        1, NCH, chunk_body, (zero, zero, zero, zero, vu0, vi0))

    accs = (a0, a1, a2, a3)
    for j in range(L):
        accs = consume(pvu, pvi, j, accs)

    accv[...] = (accs[0] + accs[1]) + (accs[2] + accs[3])
    pltpu.sync_copy(accv, part_out.at[pl.ds(wid * L, L)])


_dot_call = functools.partial(
    pl.kernel,
    out_type=jax.ShapeDtypeStruct((NW * L,), jnp.float32),
    mesh=plsc.VectorSubcoreMesh(core_axis_name="c", subcore_axis_name="s"),
    compiler_params=pltpu.CompilerParams(use_tc_tiling_on_sc=True),
    scratch_types=(
        [pltpu.VMEM((BPW,), jnp.int32)] * 2
        + [pltpu.VMEM((TS, D), jnp.float32)] * (2 * L)
        + [pltpu.VMEM((L,), jnp.float32)]
        + [pltpu.SemaphoreType.DMA] * L
    ),
)(_dot_body)


def _bias_body(ubt, ibt, uidx, iidx,
               bsum_out,
               idx_u, idx_i, bu, bi,
               sem_bu, sem_bi):
    c = lax.axis_index("c")
    s = lax.axis_index("s")
    wid = s * NC + c
    base = wid * BPW

    pltpu.sync_copy(uidx.at[pl.ds(base, BPW)], idx_u)
    pltpu.sync_copy(iidx.at[pl.ds(base, BPW)], idx_i)
    cp_bu = pltpu.async_copy(ubt.at[idx_u], bu, sem_bu)
    cp_bi = pltpu.async_copy(ibt.at[idx_i], bi, sem_bi)
    cp_bu.wait()
    cp_bi.wait()
    for j in range(BPW // L):
        sl = pl.ds(j * L, L)
        bu[sl] = bu[sl] + bi[sl]
    pltpu.sync_copy(bu, bsum_out.at[pl.ds(base, BPW)])


_bias_call = functools.partial(
    pl.kernel,
    out_type=jax.ShapeDtypeStruct((B,), jnp.float32),
    mesh=plsc.VectorSubcoreMesh(core_axis_name="c", subcore_axis_name="s"),
    compiler_params=pltpu.CompilerParams(use_tc_tiling_on_sc=False),
    scratch_types=[
        pltpu.VMEM((BPW,), jnp.int32),
        pltpu.VMEM((BPW,), jnp.int32),
        pltpu.VMEM((BPW,), jnp.float32),
        pltpu.VMEM((BPW,), jnp.float32),
        pltpu.SemaphoreType.DMA,
        pltpu.SemaphoreType.DMA,
    ],
)(_bias_body)


def _finish_body(part_ref, bsum_ref, out_ref):
    scalar = jnp.sum(part_ref[...])
    out_ref[...] = jnp.maximum(bsum_ref[...] + scalar, 0.0)


def kernel(user_emb, user_bias_tbl, item_emb, item_bias_tbl, inputs):
    idx = inputs.astype(jnp.int32).T
    uidx, iidx = idx[0], idx[1]
    partials = _dot_call(user_emb, item_emb, uidx, iidx)
    bsum = _bias_call(
        user_bias_tbl.reshape(-1), item_bias_tbl.reshape(-1), uidx, iidx)
    out = pl.pallas_call(
        _finish_body,
        out_shape=jax.ShapeDtypeStruct((B // 128, 128), jnp.float32),
    )(partials.reshape(NW, L), bsum.reshape(B // 128, 128))
    return out.reshape(B, 1)


# speedup vs baseline: 2.5023x; 1.7684x over previous
import functools

import jax
import jax.numpy as jnp
from jax import lax
from jax.experimental import pallas as pl
from jax.experimental.pallas import tpu as pltpu
from jax.experimental.pallas import tpu_sc as plsc

D = 64
B = 16384
NC = 2
NS = 16
NW = NC * NS
BPW = B // NW
L = 16
NCH = BPW // L
TS = 8
TL = 128
NBUF = 4


def _dot_body(uembT, iembT, uidx, iidx, part_out, *refs):
    idx_u, idx_i = refs[0], refs[1]
    su = refs[2:2 + NBUF]
    si = refs[2 + NBUF:2 + 2 * NBUF]
    accv = refs[2 + 2 * NBUF]
    sems = refs[3 + 2 * NBUF:]

    c = lax.axis_index("c")
    s = lax.axis_index("s")
    wid = s * NC + c
    base = wid * BPW

    pltpu.sync_copy(uidx.at[pl.ds(base, BPW)], idx_u)
    pltpu.sync_copy(iidx.at[pl.ds(base, BPW)], idx_i)

    iota = lax.broadcasted_iota(jnp.int32, (L,), 0)
    s_vec = iota % TS

    def enqueue(p_u, p_i, j):
        cu = pl.multiple_of((p_u // TL) * TL, TL)
        ci = pl.multiple_of((p_i // TL) * TL, TL)
        for t in range(TS):
            pltpu.make_async_copy(
                uembT.at[pl.ds(t * TS, TS), pl.ds(cu, TL)], su[j].at[t],
                sems[j]).start()
            pltpu.make_async_copy(
                iembT.at[pl.ds(t * TS, TS), pl.ds(ci, TL)], si[j].at[t],
                sems[j]).start()

    def wait_slot(j):
        for _ in range(2 * TS):
            pltpu.make_async_copy(
                uembT.at[pl.ds(0, TS), pl.ds(0, TL)], su[j].at[0],
                sems[j]).wait()

    def consume(pvu, pvi, j, jj, accs):
        accs = list(accs)
        wait_slot(jj)
        q_u = jnp.full((L,), pvu[j] % TL, jnp.int32)
        q_i = jnp.full((L,), pvi[j] % TL, jnp.int32)
        for t in range(TS):
            uvals = plsc.load_gather(su[jj].at[t], [s_vec, q_u])
            ivals = plsc.load_gather(si[jj].at[t], [s_vec, q_i])
            accs[t % 4] = accs[t % 4] + uvals * ivals
        return tuple(accs)

    vu0 = idx_u[pl.ds(0, L)]
    vi0 = idx_i[pl.ds(0, L)]
    for j in range(NBUF):
        enqueue(vu0[j], vi0[j], j)

    zero = jnp.zeros((L,), jnp.float32)

    def chunk_body(g, carry):
        a0, a1, a2, a3, pvu, pvi = carry
        vu = idx_u[pl.ds(g * L, L)]
        vi = idx_i[pl.ds(g * L, L)]
        accs = (a0, a1, a2, a3)
        for j in range(L):
            lane = (j + NBUF) % L
            src_u = pvu if j < NBUF else vu
            src_i = pvi if j < NBUF else vi
            accs = consume(src_u, src_i, lane, j % NBUF, accs)
            enqueue(vu[j], vi[j], j % NBUF)
        return accs + (vu, vi)

    a0, a1, a2, a3, pvu, pvi = lax.fori_loop(
        0, NCH, chunk_body, (zero, zero, zero, zero, vu0, vi0))

    accs = (a0, a1, a2, a3)
    for j in range(NBUF):
        accs = consume(pvu, pvi, j + NBUF, j, accs)

    accv[...] = ((accs[0] + accs[1]) + (accs[2] + accs[3])) * 0.5
    pltpu.sync_copy(accv, part_out.at[pl.ds(wid * L, L)])


_dot_call = functools.partial(
    pl.kernel,
    out_type=jax.ShapeDtypeStruct((NW * L,), jnp.float32),
    mesh=plsc.VectorSubcoreMesh(core_axis_name="c", subcore_axis_name="s"),
    compiler_params=pltpu.CompilerParams(
        use_tc_tiling_on_sc=True, needs_layout_passes=False),
    scratch_types=(
        [pltpu.VMEM((BPW,), jnp.int32)] * 2
        + [pltpu.VMEM((TS, TS, TL), jnp.float32)] * (2 * NBUF)
        + [pltpu.VMEM((L,), jnp.float32)]
        + [pltpu.SemaphoreType.DMA] * NBUF
    ),
)(_dot_body)


def _bias_body(ubt, ibt, uidx, iidx,
               bsum_out,
               idx_u, idx_i, bu, bi,
               sem_bu, sem_bi):
    c = lax.axis_index("c")
    s = lax.axis_index("s")
    wid = s * NC + c
    base = wid * BPW

    pltpu.sync_copy(uidx.at[pl.ds(base, BPW)], idx_u)
    pltpu.sync_copy(iidx.at[pl.ds(base, BPW)], idx_i)
    cp_bu = pltpu.async_copy(ubt.at[idx_u], bu, sem_bu)
    cp_bi = pltpu.async_copy(ibt.at[idx_i], bi, sem_bi)
    cp_bu.wait()
    cp_bi.wait()
    for j in range(BPW // L):
        sl = pl.ds(j * L, L)
        bu[sl] = bu[sl] + bi[sl]
    pltpu.sync_copy(bu, bsum_out.at[pl.ds(base, BPW)])


_bias_call = functools.partial(
    pl.kernel,
    out_type=jax.ShapeDtypeStruct((B,), jnp.float32),
    mesh=plsc.VectorSubcoreMesh(core_axis_name="c", subcore_axis_name="s"),
    compiler_params=pltpu.CompilerParams(use_tc_tiling_on_sc=False),
    scratch_types=[
        pltpu.VMEM((BPW,), jnp.int32),
        pltpu.VMEM((BPW,), jnp.int32),
        pltpu.VMEM((BPW,), jnp.float32),
        pltpu.VMEM((BPW,), jnp.float32),
        pltpu.SemaphoreType.DMA,
        pltpu.SemaphoreType.DMA,
    ],
)(_bias_body)


def _finish_body(part_ref, bsum_ref, out_ref):
    scalar = jnp.sum(part_ref[...])
    out_ref[...] = jnp.maximum(bsum_ref[...] + scalar, 0.0)


def kernel(user_emb, user_bias_tbl, item_emb, item_bias_tbl, inputs):
    idx = inputs.astype(jnp.int32).T
    uidx, iidx = idx[0], idx[1]
    partials = _dot_call(user_emb.T, item_emb.T, uidx, iidx)
    bsum = _bias_call(
        user_bias_tbl.T.reshape(-1), item_bias_tbl.T.reshape(-1),
        uidx, iidx)
    out = pl.pallas_call(
        _finish_body,
        out_shape=jax.ShapeDtypeStruct((B // 128, 128), jnp.float32),
    )(partials.reshape(NW, L), bsum.reshape(B // 128, 128))
    return out.reshape(B, 1)


# speedup vs baseline: 2.5728x; 1.0282x over previous
import functools

import jax
import jax.numpy as jnp
from jax import lax
from jax.experimental import pallas as pl
from jax.experimental.pallas import tpu as pltpu
from jax.experimental.pallas import tpu_sc as plsc

D = 64
B = 16384
NC = 2
NS = 16
NW = NC * NS
BPW = B // NW
L = 16
NCH = BPW // L
TS = 8
TL = 128
NBUF = 4


def _dot_body(uembT, iembT, uidx, iidx, part_out, *refs):
    idx_u, idx_i = refs[0], refs[1]
    su = refs[2:2 + NBUF]
    si = refs[2 + NBUF:2 + 2 * NBUF]
    accv = refs[2 + 2 * NBUF]
    sems = refs[3 + 2 * NBUF:]

    c = lax.axis_index("c")
    s = lax.axis_index("s")
    wid = s * NC + c
    base = wid * BPW

    pltpu.sync_copy(uidx.at[pl.ds(base, BPW)], idx_u)
    pltpu.sync_copy(iidx.at[pl.ds(base, BPW)], idx_i)

    iota = lax.broadcasted_iota(jnp.int32, (L,), 0)

    def enqueue(p_u, p_i, j):
        cu = pl.multiple_of((p_u // TL) * TL, TL)
        ci = pl.multiple_of((p_i // TL) * TL, TL)
        pltpu.make_async_copy(
            uembT.at[:, pl.ds(cu, TL)], su[j], sems[j]).start()
        pltpu.make_async_copy(
            iembT.at[:, pl.ds(ci, TL)], si[j], sems[j]).start()

    def wait_slot(j):
        for _ in range(2):
            pltpu.make_async_copy(
                uembT.at[:, pl.ds(0, TL)], su[j], sems[j]).wait()

    def consume(pvu, pvi, j, jj, accs):
        accs = list(accs)
        wait_slot(jj)
        q_u = jnp.full((L,), pvu[j] % TL, jnp.int32)
        q_i = jnp.full((L,), pvi[j] % TL, jnp.int32)
        for g in range(4):
            d_vec = g * L + iota
            uvals = plsc.load_gather(su[jj], [d_vec, q_u])
            ivals = plsc.load_gather(si[jj], [d_vec, q_i])
            accs[g] = accs[g] + uvals * ivals
        return tuple(accs)

    vu0 = idx_u[pl.ds(0, L)]
    vi0 = idx_i[pl.ds(0, L)]
    for j in range(NBUF):
        enqueue(vu0[j], vi0[j], j)

    zero = jnp.zeros((L,), jnp.float32)

    def chunk_body(g, carry):
        a0, a1, a2, a3, pvu, pvi = carry
        vu = idx_u[pl.ds(g * L, L)]
        vi = idx_i[pl.ds(g * L, L)]
        accs = (a0, a1, a2, a3)
        for j in range(L):
            lane = (j + NBUF) % L
            src_u = pvu if j < NBUF else vu
            src_i = pvi if j < NBUF else vi
            accs = consume(src_u, src_i, lane, j % NBUF, accs)
            enqueue(vu[j], vi[j], j % NBUF)
        return accs + (vu, vi)

    a0, a1, a2, a3, pvu, pvi = lax.fori_loop(
        0, NCH, chunk_body, (zero, zero, zero, zero, vu0, vi0))

    accs = (a0, a1, a2, a3)
    for j in range(NBUF):
        accs = consume(pvu, pvi, j + NBUF, j, accs)

    accv[...] = (accs[0] + accs[1]) + (accs[2] + accs[3])
    pltpu.sync_copy(accv, part_out.at[pl.ds(wid * L, L)])


_dot_call = functools.partial(
    pl.kernel,
    out_type=jax.ShapeDtypeStruct((NW * L,), jnp.float32),
    mesh=plsc.VectorSubcoreMesh(core_axis_name="c", subcore_axis_name="s"),
    compiler_params=pltpu.CompilerParams(
        use_tc_tiling_on_sc=True, needs_layout_passes=False),
    scratch_types=(
        [pltpu.VMEM((BPW,), jnp.int32)] * 2
        + [pltpu.VMEM((D, TL), jnp.float32)] * (2 * NBUF)
        + [pltpu.VMEM((L,), jnp.float32)]
        + [pltpu.SemaphoreType.DMA] * NBUF
    ),
)(_dot_body)


def _bias_body(ubt, ibt, uidx, iidx,
               bsum_out,
               idx_u, idx_i, bu, bi,
               sem_bu, sem_bi):
    c = lax.axis_index("c")
    s = lax.axis_index("s")
    wid = s * NC + c
    base = wid * BPW

    pltpu.sync_copy(uidx.at[pl.ds(base, BPW)], idx_u)
    pltpu.sync_copy(iidx.at[pl.ds(base, BPW)], idx_i)
    cp_bu = pltpu.async_copy(ubt.at[idx_u], bu, sem_bu)
    cp_bi = pltpu.async_copy(ibt.at[idx_i], bi, sem_bi)
    cp_bu.wait()
    cp_bi.wait()
    for j in range(BPW // L):
        sl = pl.ds(j * L, L)
        bu[sl] = bu[sl] + bi[sl]
    pltpu.sync_copy(bu, bsum_out.at[pl.ds(base, BPW)])


_bias_call = functools.partial(
    pl.kernel,
    out_type=jax.ShapeDtypeStruct((B,), jnp.float32),
    mesh=plsc.VectorSubcoreMesh(core_axis_name="c", subcore_axis_name="s"),
    compiler_params=pltpu.CompilerParams(use_tc_tiling_on_sc=False),
    scratch_types=[
        pltpu.VMEM((BPW,), jnp.int32),
        pltpu.VMEM((BPW,), jnp.int32),
        pltpu.VMEM((BPW,), jnp.float32),
        pltpu.VMEM((BPW,), jnp.float32),
        pltpu.SemaphoreType.DMA,
        pltpu.SemaphoreType.DMA,
    ],
)(_bias_body)


def _finish_body(part_ref, bsum_ref, out_ref):
    scalar = jnp.sum(part_ref[...])
    out_ref[...] = jnp.maximum(bsum_ref[...] + scalar, 0.0)


def kernel(user_emb, user_bias_tbl, item_emb, item_bias_tbl, inputs):
    idx = inputs.astype(jnp.int32).T
    uidx, iidx = idx[0], idx[1]
    partials = _dot_call(user_emb.T, item_emb.T, uidx, iidx)
    bsum = _bias_call(
        user_bias_tbl.T.reshape(-1), item_bias_tbl.T.reshape(-1),
        uidx, iidx)
    out = pl.pallas_call(
        _finish_body,
        out_shape=jax.ShapeDtypeStruct((B // 128, 128), jnp.float32),
    )(partials.reshape(NW, L), bsum.reshape(B // 128, 128))
    return out.reshape(B, 1)
